# baseline (speedup 1.0000x reference)
import jax
import jax.numpy as jnp
from jax import lax
from jax.experimental import pallas as pl
from jax.experimental.pallas import tpu as pltpu
from jax.experimental.pallas import tpu_sc as plsc

_N0, _N1, _N2 = 10000, 5000, 2000
_D = 128
_E0, _E1, _E2 = 640000, 200000, 100000
_LAYERS = 2

_NC, _NS = 2, 16
_NW = _NC * _NS
_CHUNK = 128

_CPW = 162
_EW = _CPW * _CHUNK
_EP = _NW * _EW
_CH = _EP // _CHUNK
_CPT = _EP // (_NS * _CHUNK)
_NPAD = 10112
_ZROWS = _NPAD // _NS

_CW1 = 49
_EW1 = _CW1 * _CHUNK
_E1P = _NW * _EW1
_CW2 = 25
_EW2 = _CW2 * _CHUNK
_E2P = _NW * _EW2
_NB1 = 5008
_NB2 = 2016
_NBT = 8192
_SLICE = _NBT // _NS

_mesh = plsc.VectorSubcoreMesh(core_axis_name="c", subcore_axis_name="s",
                               num_cores=_NC, num_subcores=_NS)
_mesh1 = plsc.VectorSubcoreMesh(core_axis_name="c", subcore_axis_name="s",
                                num_cores=1, num_subcores=_NS)


def _sc_segsum_body(x_hbm, gidx_hbm, sidx_hbm, out_hbm, gi, si, data, acc,
                    sem0, sem1, sem2):
    sid = lax.axis_index("s")

    def zrow(i, carry):
        for j in range(_D // 16):
            data[0, i, pl.ds(j * 16, 16)] = jnp.zeros((16,), jnp.float32)
        return carry

    lax.fori_loop(0, _CHUNK, zrow, 0)
    for k in range(_ZROWS // _CHUNK):
        pltpu.sync_copy(data.at[0], acc.at[pl.ds(sid * _ZROWS + k * _CHUNK, _CHUNK)])
    pltpu.sync_copy(data.at[0].at[pl.ds(0, _ZROWS % _CHUNK)],
                    acc.at[pl.ds(sid * _ZROWS + (_ZROWS // _CHUNK) * _CHUNK,
                                 _ZROWS % _CHUNK)])
    plsc.subcore_barrier()

    sems = [sem0, sem1, sem2]
    base = sid * _CPT

    def load_and_fire(c, j):
        pltpu.sync_copy(gidx_hbm.at[pl.ds((base + c) * _CHUNK, _CHUNK)], gi.at[j])
        pltpu.async_copy(x_hbm.at[gi.at[j]], data.at[j], sems[j])

    def drain_scatter(c, j):
        pltpu.make_async_copy(x_hbm.at[gi.at[j]], data.at[j], sems[j]).wait()
        pltpu.sync_copy(sidx_hbm.at[pl.ds((base + c) * _CHUNK, _CHUNK)], si)
        pltpu.sync_copy(data.at[j], acc.at[si], add=True)

    for j in range(3):
        load_and_fire(j, j)

    def body(t, carry):
        for j in range(3):
            c = 3 * t + j
            drain_scatter(c, j)
            load_and_fire(c + 3, j)
        return carry

    lax.fori_loop(0, _CPT // 3 - 1, body, 0)
    for j in range(3):
        drain_scatter(_CPT - 3 + j, j)
    plsc.subcore_barrier()
    pltpu.sync_copy(acc.at[pl.ds(sid * _ZROWS, _ZROWS)],
                    out_hbm.at[pl.ds(sid * _ZROWS, _ZROWS)])


_sc_segsum = pl.kernel(
    _sc_segsum_body,
    out_type=jax.ShapeDtypeStruct((_NPAD, _D), jnp.float32),
    mesh=_mesh1,
    scratch_types=[
        pltpu.VMEM((3, _CHUNK), jnp.int32),
        pltpu.VMEM((_CHUNK,), jnp.int32),
        pltpu.VMEM((3, _CHUNK, _D), jnp.float32),
        pltpu.VMEM_SHARED((_NPAD, _D), jnp.float32),
        pltpu.SemaphoreType.DMA,
        pltpu.SemaphoreType.DMA,
        pltpu.SemaphoreType.DMA,
    ],
)


def _sc_deg_body(r1_hbm, r2_hbm, out_hbm, idx_v, buf2d, red, tmp, obuf, slots):
    cid = lax.axis_index("c")
    sid = lax.axis_index("s")
    w = cid * _NS + sid
    lanes = lax.iota(jnp.int32, 16)
    ones = jnp.ones((16,), jnp.float32)

    def zcols(c, carry):
        for l in range(16):
            buf2d[pl.ds(l * _NB1 + c * 16, 16)] = jnp.zeros((16,), jnp.float32)
        return carry

    def scatter_chunk(hbm, base):
        pltpu.sync_copy(hbm.at[pl.ds(base, _CHUNK)], idx_v)
        for k in range(_CHUNK // 16):
            gi = idx_v[pl.ds(k * 16, 16)]
            plsc.addupdate_scatter(buf2d, [lanes * _NB1 + gi], ones)

    def reduce_cols(c, out_base):
        s = buf2d[pl.ds(c * 16, 16)]
        for l in range(1, 16):
            s = s + buf2d[pl.ds(l * _NB1 + c * 16, 16)]
        red[pl.ds(out_base + c * 16, 16)] = s

    lax.fori_loop(0, _NB1 // 16, zcols, 0)
    lax.fori_loop(0, _CW1, lambda c, k: (scatter_chunk(r1_hbm, w * _EW1 + c * _CHUNK), k)[1], 0)
    lax.fori_loop(0, _NB1 // 16, lambda c, k: (reduce_cols(c, 0), k)[1], 0)

    lax.fori_loop(0, _NB2 // 16, zcols, 0)
    lax.fori_loop(0, _CW2, lambda c, k: (scatter_chunk(r2_hbm, w * _EW2 + c * _CHUNK), k)[1], 0)
    lax.fori_loop(0, _NB2 // 16, lambda c, k: (reduce_cols(c, _NB1), k)[1], 0)

    def ztail(c, carry):
        red[pl.ds(_NB1 + _NB2 + c * 16, 16)] = jnp.zeros((16,), jnp.float32)
        return carry

    lax.fori_loop(0, (_NBT - _NB1 - _NB2) // 16, ztail, 0)

    pltpu.sync_copy(red, slots.at[sid])
    plsc.subcore_barrier()
    for l in range(16):
        pltpu.sync_copy(slots.at[l, pl.ds(sid * _SLICE, _SLICE)],
                        tmp.at[pl.ds(l * _SLICE, _SLICE)])

    def reduce_slice(c, carry):
        s = tmp[pl.ds(c * 16, 16)]
        for l in range(1, 16):
            s = s + tmp[pl.ds(l * _SLICE + c * 16, 16)]
        obuf[pl.ds(c * 16, 16)] = s
        return carry

    lax.fori_loop(0, _SLICE // 16, reduce_slice, 0)
    pltpu.sync_copy(obuf, out_hbm.at[cid, 0, pl.ds(sid * _SLICE, _SLICE)])


_sc_deg = pl.kernel(
    _sc_deg_body,
    out_type=jax.ShapeDtypeStruct((_NC, 1, _NBT), jnp.float32),
    mesh=_mesh,
    scratch_types=[
        pltpu.VMEM((_CHUNK,), jnp.int32),
        pltpu.VMEM((16 * _NB1,), jnp.float32),
        pltpu.VMEM((_NBT,), jnp.float32),
        pltpu.VMEM((16 * _SLICE,), jnp.float32),
        pltpu.VMEM((_SLICE,), jnp.float32),
        pltpu.VMEM_SHARED((16, _NBT), jnp.float32),
    ],
    compiler_params=pltpu.CompilerParams(needs_layout_passes=False),
)


_BLK = 2000


def _tc_mm_body(p_ref, w1_ref, b1_ref, w2_ref, o_ref):
    h = p_ref[...]
    x1 = jnp.maximum(
        jnp.dot(h, w1_ref[...], preferred_element_type=jnp.float32) + b1_ref[...], 0.0)
    o_ref[...] = jnp.dot(x1, w2_ref[...], preferred_element_type=jnp.float32)


_tc_mm = pl.pallas_call(
    _tc_mm_body,
    grid=(_N0 // _BLK,),
    in_specs=[
        pl.BlockSpec((_BLK, _D), lambda i: (i, 0)),
        pl.BlockSpec((_D, _D), lambda i: (0, 0)),
        pl.BlockSpec((1, _D), lambda i: (0, 0)),
        pl.BlockSpec((_D, _D), lambda i: (0, 0)),
    ],
    out_specs=pl.BlockSpec((_BLK, _D), lambda i: (i, 0)),
    out_shape=jax.ShapeDtypeStruct((_N0, _D), jnp.float32),
)


def _tc_bias_relu_body(p_ref, b2_ref, o_ref):
    o_ref[...] = jnp.maximum(p_ref[...] + b2_ref[...], 0.0)


_tc_bias_relu = pl.pallas_call(
    _tc_bias_relu_body,
    grid=(_N0 // _BLK,),
    in_specs=[
        pl.BlockSpec((_BLK, _D), lambda i: (i, 0)),
        pl.BlockSpec((1, _D), lambda i: (0, 0)),
    ],
    out_specs=pl.BlockSpec((_BLK, _D), lambda i: (i, 0)),
    out_shape=jax.ShapeDtypeStruct((_N0, _D), jnp.float32),
)


def _tc_routes_body(d1_ref, d2_ref, b1a_ref, w2a_ref, b2a_ref,
                    b1b_ref, w2b_ref, b2b_ref, o1_ref, o2_ref):
    va = jnp.dot(jnp.maximum(b1a_ref[...], 0.0), w2a_ref[...],
                 preferred_element_type=jnp.float32)
    d1 = d1_ref[:, 0:1] + d1_ref[:, 1:2]
    o1_ref[...] = jnp.maximum(d1 * va + b2a_ref[...], 0.0)
    vb = jnp.dot(jnp.maximum(b1b_ref[...], 0.0), w2b_ref[...],
                 preferred_element_type=jnp.float32)
    d2 = d2_ref[:, 0:1] + d2_ref[:, 1:2]
    o2_ref[...] = jnp.maximum(d2 * vb + b2b_ref[...], 0.0)


_tc_routes = pl.pallas_call(
    _tc_routes_body,
    out_shape=(
        jax.ShapeDtypeStruct((_N1, _D), jnp.float32),
        jax.ShapeDtypeStruct((_N2, _D), jnp.float32),
    ),
)


def kernel(x_0, x_1, x_2, adj0_index, inc1_index, inc2_index, cell_statistics,
           W1, b1, W2, b2):
    del x_1, x_2, cell_statistics
    rows0 = adj0_index[0]
    cols0 = adj0_index[1]
    padg = jnp.zeros((_EP - _E0,), jnp.int32)
    pads = jnp.full((_EP - _E0,), _N0, jnp.int32)
    g1 = jnp.concatenate([rows0, padg])
    s1 = jnp.concatenate([cols0, pads])
    g2 = jnp.concatenate([cols0, padg])
    s2 = jnp.concatenate([rows0, pads])
    r1 = jnp.concatenate([inc1_index[0], jnp.full((_E1P - _E1,), _N1, jnp.int32)])
    r2 = jnp.concatenate([inc2_index[0], jnp.full((_E2P - _E2,), _N2, jnp.int32)])

    x = x_0
    for l in range(_LAYERS):
        p_he = _sc_segsum(x, g1, s1)
        y1 = _tc_mm(p_he, W1[l, 0], b1[l, 0].reshape(1, _D), W2[l, 0])
        p_agg = _sc_segsum(y1, g2, s2)
        x = _tc_bias_relu(p_agg, b2[l, 0].reshape(1, _D))
    out_0 = x

    dp = _sc_deg(r1, r2)
    d1t = dp[:, 0, :_N1].T
    d2t = dp[:, 0, _NB1:_NB1 + _N2].T
    lw = _LAYERS - 1
    out_1, out_2 = _tc_routes(
        d1t, d2t,
        b1[lw, 1].reshape(1, _D), W2[lw, 1], b2[lw, 1].reshape(1, _D),
        b1[lw, 2].reshape(1, _D), W2[lw, 2], b2[lw, 2].reshape(1, _D))
    return out_0, out_1, out_2


# speedup vs baseline: 1.2400x; 1.2400x over previous
import jax
import jax.numpy as jnp
from jax import lax
from jax.experimental import pallas as pl
from jax.experimental.pallas import tpu as pltpu
from jax.experimental.pallas import tpu_sc as plsc

_N0, _N1, _N2 = 10000, 5000, 2000
_D = 128
_E0, _E1, _E2 = 640000, 200000, 100000
_LAYERS = 2

_NC, _NS = 2, 16
_NW = _NC * _NS
_CHUNK = 128

_CPW = 160
_EW = _CPW * _CHUNK
_EP = _NW * _EW
_NPAD = 10240
_ZROWS = _NPAD // _NS

_CW1 = 49
_EW1 = _CW1 * _CHUNK
_E1P = _NW * _EW1
_CW2 = 25
_EW2 = _CW2 * _CHUNK
_E2P = _NW * _EW2
_NB1 = 5008
_NB2 = 2016
_NBT = 8192
_SLICE = _NBT // _NS

_mesh = plsc.VectorSubcoreMesh(core_axis_name="c", subcore_axis_name="s",
                               num_cores=_NC, num_subcores=_NS)


def _sc_segsum_body(x_hbm, gidx_hbm, sidx_hbm, out_hbm, gi_v, si_v, data, acc, gsem):
    cid = lax.axis_index("c")
    sid = lax.axis_index("s")
    w = cid * _NS + sid

    def zrow(i, carry):
        for j in range(_D // 16):
            data[0, i, pl.ds(j * 16, 16)] = jnp.zeros((16,), jnp.float32)
        return carry

    lax.fori_loop(0, _CHUNK, zrow, 0)
    for k in range(_ZROWS // _CHUNK):
        pltpu.sync_copy(data.at[0], acc.at[pl.ds(sid * _ZROWS + k * _CHUNK, _CHUNK)])
    plsc.subcore_barrier()

    def chunk(c, carry):
        base = w * _EW + c * _CHUNK
        pltpu.sync_copy(gidx_hbm.at[pl.ds(base, _CHUNK)], gi_v)
        pltpu.sync_copy(sidx_hbm.at[pl.ds(base, _CHUNK)], si_v)
        pltpu.async_copy(x_hbm.at[gi_v], data.at[0], gsem).wait()
        pltpu.sync_copy(data.at[0], acc.at[si_v], add=True)
        return carry

    lax.fori_loop(0, _CPW, chunk, 0)
    plsc.subcore_barrier()
    pltpu.sync_copy(acc.at[pl.ds(sid * _ZROWS, _ZROWS)],
                    out_hbm.at[cid, pl.ds(sid * _ZROWS, _ZROWS)])


_sc_segsum = pl.kernel(
    _sc_segsum_body,
    out_type=jax.ShapeDtypeStruct((_NC, _NPAD, _D), jnp.float32),
    mesh=_mesh,
    scratch_types=[
        pltpu.VMEM((_CHUNK,), jnp.int32),
        pltpu.VMEM((_CHUNK,), jnp.int32),
        pltpu.VMEM((1, _CHUNK, _D), jnp.float32),
        pltpu.VMEM_SHARED((_NPAD, _D), jnp.float32),
        pltpu.SemaphoreType.DMA,
    ],
)


def _sc_deg_body(r1_hbm, r2_hbm, out_hbm, idx_v, buf2d, red, tmp, obuf, slots):
    cid = lax.axis_index("c")
    sid = lax.axis_index("s")
    w = cid * _NS + sid
    lanes = lax.iota(jnp.int32, 16)
    ones = jnp.ones((16,), jnp.float32)

    def zcols(c, carry):
        for l in range(16):
            buf2d[pl.ds(l * _NB1 + c * 16, 16)] = jnp.zeros((16,), jnp.float32)
        return carry

    def scatter_chunk(hbm, base):
        pltpu.sync_copy(hbm.at[pl.ds(base, _CHUNK)], idx_v)
        for k in range(_CHUNK // 16):
            gi = idx_v[pl.ds(k * 16, 16)]
            plsc.addupdate_scatter(buf2d, [lanes * _NB1 + gi], ones)

    def reduce_cols(c, out_base):
        s = buf2d[pl.ds(c * 16, 16)]
        for l in range(1, 16):
            s = s + buf2d[pl.ds(l * _NB1 + c * 16, 16)]
        red[pl.ds(out_base + c * 16, 16)] = s

    lax.fori_loop(0, _NB1 // 16, zcols, 0)
    lax.fori_loop(0, _CW1, lambda c, k: (scatter_chunk(r1_hbm, w * _EW1 + c * _CHUNK), k)[1], 0)
    lax.fori_loop(0, _NB1 // 16, lambda c, k: (reduce_cols(c, 0), k)[1], 0)

    lax.fori_loop(0, _NB2 // 16, zcols, 0)
    lax.fori_loop(0, _CW2, lambda c, k: (scatter_chunk(r2_hbm, w * _EW2 + c * _CHUNK), k)[1], 0)
    lax.fori_loop(0, _NB2 // 16, lambda c, k: (reduce_cols(c, _NB1), k)[1], 0)

    def ztail(c, carry):
        red[pl.ds(_NB1 + _NB2 + c * 16, 16)] = jnp.zeros((16,), jnp.float32)
        return carry

    lax.fori_loop(0, (_NBT - _NB1 - _NB2) // 16, ztail, 0)

    pltpu.sync_copy(red, slots.at[sid])
    plsc.subcore_barrier()
    for l in range(16):
        pltpu.sync_copy(slots.at[l, pl.ds(sid * _SLICE, _SLICE)],
                        tmp.at[pl.ds(l * _SLICE, _SLICE)])

    def reduce_slice(c, carry):
        s = tmp[pl.ds(c * 16, 16)]
        for l in range(1, 16):
            s = s + tmp[pl.ds(l * _SLICE + c * 16, 16)]
        obuf[pl.ds(c * 16, 16)] = s
        return carry

    lax.fori_loop(0, _SLICE // 16, reduce_slice, 0)
    pltpu.sync_copy(obuf, out_hbm.at[cid, 0, pl.ds(sid * _SLICE, _SLICE)])


_sc_deg = pl.kernel(
    _sc_deg_body,
    out_type=jax.ShapeDtypeStruct((_NC, 1, _NBT), jnp.float32),
    mesh=_mesh,
    scratch_types=[
        pltpu.VMEM((_CHUNK,), jnp.int32),
        pltpu.VMEM((16 * _NB1,), jnp.float32),
        pltpu.VMEM((_NBT,), jnp.float32),
        pltpu.VMEM((16 * _SLICE,), jnp.float32),
        pltpu.VMEM((_SLICE,), jnp.float32),
        pltpu.VMEM_SHARED((16, _NBT), jnp.float32),
    ],
    compiler_params=pltpu.CompilerParams(needs_layout_passes=False),
)


_BLK = 2000


def _tc_mm_body(p_ref, w1_ref, b1_ref, w2_ref, o_ref):
    h = p_ref[0] + p_ref[1]
    x1 = jnp.maximum(
        jnp.dot(h, w1_ref[...], preferred_element_type=jnp.float32) + b1_ref[...], 0.0)
    o_ref[...] = jnp.dot(x1, w2_ref[...], preferred_element_type=jnp.float32)


_tc_mm = pl.pallas_call(
    _tc_mm_body,
    grid=(_N0 // _BLK,),
    in_specs=[
        pl.BlockSpec((_NC, _BLK, _D), lambda i: (0, i, 0)),
        pl.BlockSpec((_D, _D), lambda i: (0, 0)),
        pl.BlockSpec((1, _D), lambda i: (0, 0)),
        pl.BlockSpec((_D, _D), lambda i: (0, 0)),
    ],
    out_specs=pl.BlockSpec((_BLK, _D), lambda i: (i, 0)),
    out_shape=jax.ShapeDtypeStruct((_N0, _D), jnp.float32),
)


def _tc_bias_relu_body(p_ref, b2_ref, o_ref):
    o_ref[...] = jnp.maximum(p_ref[0] + p_ref[1] + b2_ref[...], 0.0)


_tc_bias_relu = pl.pallas_call(
    _tc_bias_relu_body,
    grid=(_N0 // _BLK,),
    in_specs=[
        pl.BlockSpec((_NC, _BLK, _D), lambda i: (0, i, 0)),
        pl.BlockSpec((1, _D), lambda i: (0, 0)),
    ],
    out_specs=pl.BlockSpec((_BLK, _D), lambda i: (i, 0)),
    out_shape=jax.ShapeDtypeStruct((_N0, _D), jnp.float32),
)


def _tc_routes_body(d1_ref, d2_ref, b1a_ref, w2a_ref, b2a_ref,
                    b1b_ref, w2b_ref, b2b_ref, o1_ref, o2_ref):
    va = jnp.dot(jnp.maximum(b1a_ref[...], 0.0), w2a_ref[...],
                 preferred_element_type=jnp.float32)
    d1 = d1_ref[:, 0:1] + d1_ref[:, 1:2]
    o1_ref[...] = jnp.maximum(d1 * va + b2a_ref[...], 0.0)
    vb = jnp.dot(jnp.maximum(b1b_ref[...], 0.0), w2b_ref[...],
                 preferred_element_type=jnp.float32)
    d2 = d2_ref[:, 0:1] + d2_ref[:, 1:2]
    o2_ref[...] = jnp.maximum(d2 * vb + b2b_ref[...], 0.0)


_tc_routes = pl.pallas_call(
    _tc_routes_body,
    out_shape=(
        jax.ShapeDtypeStruct((_N1, _D), jnp.float32),
        jax.ShapeDtypeStruct((_N2, _D), jnp.float32),
    ),
)


def kernel(x_0, x_1, x_2, adj0_index, inc1_index, inc2_index, cell_statistics,
           W1, b1, W2, b2):
    del x_1, x_2, cell_statistics
    rows0 = adj0_index[0]
    cols0 = adj0_index[1]
    padg = jnp.zeros((_EP - _E0,), jnp.int32)
    pads = jnp.full((_EP - _E0,), _N0, jnp.int32)
    g1 = jnp.concatenate([rows0, padg])
    s1 = jnp.concatenate([cols0, pads])
    g2 = jnp.concatenate([cols0, padg])
    s2 = jnp.concatenate([rows0, pads])
    r1 = jnp.concatenate([inc1_index[0], jnp.full((_E1P - _E1,), _N1, jnp.int32)])
    r2 = jnp.concatenate([inc2_index[0], jnp.full((_E2P - _E2,), _N2, jnp.int32)])

    x = x_0
    for l in range(_LAYERS):
        p_he = _sc_segsum(x, g1, s1)
        y1 = _tc_mm(p_he, W1[l, 0], b1[l, 0].reshape(1, _D), W2[l, 0])
        p_agg = _sc_segsum(y1, g2, s2)
        x = _tc_bias_relu(p_agg, b2[l, 0].reshape(1, _D))
    out_0 = x

    dp = _sc_deg(r1, r2)
    d1t = dp[:, 0, :_N1].T
    d2t = dp[:, 0, _NB1:_NB1 + _N2].T
    lw = _LAYERS - 1
    out_1, out_2 = _tc_routes(
        d1t, d2t,
        b1[lw, 1].reshape(1, _D), W2[lw, 1], b2[lw, 1].reshape(1, _D),
        b1[lw, 2].reshape(1, _D), W2[lw, 2], b2[lw, 2].reshape(1, _D))
    return out_0, out_1, out_2
